# Initial kernel scaffold; baseline (speedup 1.0000x reference)
import jax
import jax.numpy as jnp
from jax.experimental import pallas as pl


def kernel(x, atom_fea):
    raise NotImplementedError("write your pallas kernel here")


# speedup vs baseline: 2.3779x; 2.3779x over previous
import functools

import jax
import jax.numpy as jnp
from jax import lax
from jax.experimental import pallas as pl
from jax.experimental.pallas import tpu as pltpu
from jax.experimental.pallas import tpu_sc as plsc

VOCAB = 120
EMBED_DIM = 200
CHUNK = 128


def _sc_gather(idx3, table, B):
    info = plsc.get_sparse_core_info()
    NC, NS = info.num_cores, info.num_subcores
    NW = NC * NS
    n_chunks = idx3.shape[1]
    b_per_w = n_chunks * CHUNK
    mesh = plsc.VectorSubcoreMesh(core_axis_name="c", subcore_axis_name="s")

    @functools.partial(
        pl.kernel,
        mesh=mesh,
        compiler_params=pltpu.CompilerParams(use_tc_tiling_on_sc=False),
        out_type=jax.ShapeDtypeStruct((B, EMBED_DIM), jnp.float32),
        scratch_types=[
            pltpu.VMEM((n_chunks, CHUNK), jnp.int32),
            pltpu.VMEM((CHUNK, EMBED_DIM), jnp.float32),
            pltpu.VMEM((CHUNK, EMBED_DIM), jnp.float32),
            pltpu.SemaphoreType.DMA,
            pltpu.SemaphoreType.DMA,
            pltpu.SemaphoreType.DMA,
            pltpu.SemaphoreType.DMA,
        ],
    )
    def k(idx_hbm, table_hbm, out_hbm, idx_v, rows0, rows1, g0, g1, w0, w1):
        wid = lax.axis_index("s") * NC + lax.axis_index("c")
        base = wid * b_per_w
        pltpu.sync_copy(idx_hbm.at[wid], idx_v)
        rows = (rows0, rows1)
        gsem = (g0, g1)
        wsem = (w0, w1)

        pltpu.async_copy(table_hbm.at[idx_v.at[0]], rows0, g0)

        def body(i, carry):
            for s in range(2):
                j = 2 * i + s
                cur, nxt = rows[s], rows[1 - s]
                @pl.when(j + 1 < n_chunks)
                def _():
                    @pl.when(j >= 1)
                    def _():
                        pltpu.make_async_copy(
                            rows[1 - s], out_hbm.at[pl.ds(0, CHUNK)], wsem[1 - s]
                        ).wait()

                    pltpu.async_copy(
                        table_hbm.at[idx_v.at[j + 1]], nxt, gsem[1 - s]
                    )

                pltpu.make_async_copy(
                    table_hbm.at[idx_v.at[j]], cur, gsem[s]
                ).wait()
                pltpu.async_copy(
                    cur, out_hbm.at[pl.ds(base + j * CHUNK, CHUNK)], wsem[s]
                )
            return carry

        lax.fori_loop(0, n_chunks // 2, body, 0, unroll=False)
        pltpu.make_async_copy(
            rows0, out_hbm.at[pl.ds(0, CHUNK)], w0
        ).wait()
        pltpu.make_async_copy(
            rows1, out_hbm.at[pl.ds(0, CHUNK)], w1
        ).wait()

    return k(idx3, table)


def kernel(x, atom_fea):
    orig_shape = x.shape
    B = x.size
    info = plsc.get_sparse_core_info()
    NW = info.num_cores * info.num_subcores
    n_chunks = B // (NW * CHUNK)
    idx3 = x.astype(jnp.int32).reshape(NW, n_chunks, CHUNK)
    out = _sc_gather(idx3, atom_fea, B)
    return out.reshape(*orig_shape, EMBED_DIM)
